# Initial kernel scaffold; baseline (speedup 1.0000x reference)
import jax
import jax.numpy as jnp
from jax.experimental import pallas as pl


def kernel(hidden, edge_index, edge_attr, edge_time_emb, boundary_condition, alpha, r1, i1, rb1, ib1, fre_W, fre_b, comb_W, comb_b, lin_W, lin_b, ln_g, ln_b):
    raise NotImplementedError("write your pallas kernel here")


# speedup vs baseline: 10.7505x; 10.7505x over previous
import functools

import numpy as np
import jax
import jax.numpy as jnp
from jax import lax
from jax.experimental import pallas as pl
from jax.experimental.pallas import tpu as pltpu
from jax.experimental.pallas import tpu_sc as plsc

_N_NODES = 10000
_N_EDGES = 160000
_EMB = 128
_IN = 160
_NK = 3
_LAMBD = 0.01
_Y = _NK * _EMB

_NW = 32
_CHUNK = 128
_E_PAD = 163840
_CH_PER_W = _E_PAD // (_NW * _CHUNK)
_EDGES_PER_W = _E_PAD // _NW
_STAGE = 8
_STRIPE = 624
_TAIL = _N_NODES - 16 * _STRIPE

_EB = 2048
_NEB = _E_PAD // _EB
_NB = 2000
_NNB = _N_NODES // _NB


def _dft_consts():
    j1 = np.arange(_IN)
    a1 = 2.0 * np.pi * np.outer(j1, j1) / _IN
    C = np.cos(a1).astype(np.float32)
    S = (-np.sin(a1)).astype(np.float32)
    j2 = np.arange(_Y)
    a2 = 2.0 * np.pi * np.outer(j2, j2) / _Y
    A = (np.cos(a2) / _Y).astype(np.float32)
    B = (-np.sin(a2) / _Y).astype(np.float32)
    return C, S, A, B


_C, _S, _A, _B = _dft_consts()
_FAC = np.array([[(2 * k + 1) / (2.0 * _NK)] for k in range(_NK)], np.float32)


def _prep_body(r1, i1, freW, combE, freb, combb, alpha, C, S, A, B,
               mr_o, mi_o, wr2_o, wi2_o, b2_o, clo_o, chi_o):
    hp = lax.Precision.HIGHEST
    R = jnp.concatenate([r1[0], r1[1], r1[2]], axis=1)
    I = jnp.concatenate([i1[0], i1[1], i1[2]], axis=1)
    Cm = C[...]
    Sm = S[...]
    mr_o[...] = (jnp.dot(Cm, R, precision=hp, preferred_element_type=jnp.float32)
                 - jnp.dot(Sm, I, precision=hp, preferred_element_type=jnp.float32))
    mi_o[...] = (jnp.dot(Sm, R, precision=hp, preferred_element_type=jnp.float32)
                 + jnp.dot(Cm, I, precision=hp, preferred_element_type=jnp.float32))
    W2 = jnp.dot(freW[...], combE[...], precision=hp,
                 preferred_element_type=jnp.float32)
    wr2_o[...] = jnp.dot(A[...], W2, precision=hp,
                         preferred_element_type=jnp.float32)
    wi2_o[...] = jnp.dot(B[...], W2, precision=hp,
                         preferred_element_type=jnp.float32)
    b2_o[...] = (jnp.dot(freb[...], combE[...], precision=hp,
                         preferred_element_type=jnp.float32) + combb[...])
    av = alpha[...]
    ik = lax.broadcasted_iota(jnp.int32, (_NK, 1), 0).astype(jnp.float32)
    fac = (2.0 * ik + 1.0) / (2.0 * _NK)
    half = 1.0 / (2.0 * _NK)
    clo_o[...] = av * fac - half / av
    chi_o[...] = av * fac + half / av


def _prep(r1, i1, freW, combE, freb, combb, alpha):
    f32 = jnp.float32
    outs = (
        jax.ShapeDtypeStruct((_IN, _Y), f32),
        jax.ShapeDtypeStruct((_IN, _Y), f32),
        jax.ShapeDtypeStruct((_Y, _EMB), f32),
        jax.ShapeDtypeStruct((_Y, _EMB), f32),
        jax.ShapeDtypeStruct((1, _EMB), f32),
        jax.ShapeDtypeStruct((_NK, 1), f32),
        jax.ShapeDtypeStruct((_NK, 1), f32),
    )
    return pl.pallas_call(_prep_body, out_shape=outs)(
        r1, i1, freW, combE, freb, combb, alpha,
        jnp.asarray(_C), jnp.asarray(_S), jnp.asarray(_A), jnp.asarray(_B))


@functools.cache
def _sc_kernels():
    mesh = plsc.VectorSubcoreMesh(core_axis_name="c", subcore_axis_name="s")

    @functools.partial(
        pl.kernel,
        out_type=jax.ShapeDtypeStruct((_E_PAD, _EMB), jnp.float32),
        mesh=mesh,
        scratch_types=[
            pltpu.VMEM((_CHUNK,), jnp.int32),
            pltpu.VMEM((_CHUNK, _EMB), jnp.float32),
            pltpu.SemaphoreType.DMA,
        ],
    )
    def sc_gather(table_hbm, src_hbm, out_hbm, idx_v, rows_v, sem):
        c = lax.axis_index("c")
        s = lax.axis_index("s")
        wid = s * 2 + c

        def body(i, carry):
            base = pl.multiple_of((wid * _CH_PER_W + i) * _CHUNK, _CHUNK)
            pltpu.sync_copy(src_hbm.at[pl.ds(base, _CHUNK)], idx_v)
            pltpu.async_copy(table_hbm.at[idx_v], rows_v, sem).wait()
            pltpu.sync_copy(rows_v, out_hbm.at[pl.ds(base, _CHUNK)])
            return carry

        lax.fori_loop(0, _CH_PER_W, body, 0)

    @functools.partial(
        pl.kernel,
        out_type=jax.ShapeDtypeStruct((2, _N_NODES, _EMB), jnp.float32),
        mesh=mesh,
        scratch_types=[
            pltpu.VMEM((_STAGE, _CHUNK), jnp.int32),
            pltpu.VMEM((_CHUNK, _EMB), jnp.float32),
            pltpu.VMEM_SHARED((_N_NODES, _EMB), jnp.float32),
        ],
    )
    def sc_scatter(msg_hbm, dst2d_hbm, zeros_hbm, out_hbm, idx_v, vals_v,
                   acc_sh):
        c = lax.axis_index("c")
        s = lax.axis_index("s")
        wid = s * 2 + c
        srow = pl.multiple_of(s * _STRIPE, 8)
        pltpu.sync_copy(zeros_hbm.at[pl.ds(0, _STRIPE)],
                        acc_sh.at[pl.ds(srow, _STRIPE)])

        @pl.when(s == 15)
        def _():
            pltpu.sync_copy(zeros_hbm.at[pl.ds(0, _TAIL)],
                            acc_sh.at[pl.ds(16 * _STRIPE, _TAIL)])

        plsc.subcore_barrier()

        def body(t, carry):
            edge0 = pl.multiple_of(
                wid * _EDGES_PER_W + t * (_STAGE * _CHUNK), _STAGE * _CHUNK)
            pltpu.sync_copy(
                dst2d_hbm.at[pl.ds(pl.multiple_of(edge0 // _CHUNK, _STAGE),
                                   _STAGE)], idx_v)
            for j in range(_STAGE):
                base = pl.multiple_of(edge0 + j * _CHUNK, _CHUNK)
                pltpu.sync_copy(msg_hbm.at[pl.ds(base, _CHUNK)], vals_v)
                pltpu.sync_copy(vals_v, acc_sh.at[idx_v.at[j]], add=True)
            return carry

        lax.fori_loop(0, _EDGES_PER_W // (_STAGE * _CHUNK), body, 0)
        plsc.subcore_barrier()
        pltpu.sync_copy(acc_sh.at[pl.ds(srow, _STRIPE)],
                        out_hbm.at[c, pl.ds(srow, _STRIPE)])

        @pl.when(s == 15)
        def _():
            pltpu.sync_copy(acc_sh.at[pl.ds(16 * _STRIPE, _TAIL)],
                            out_hbm.at[c, pl.ds(16 * _STRIPE, _TAIL)])

    return sc_gather, sc_scatter


def _sc_gather(table, src_p):
    return _sc_kernels()[0](table, src_p)


def _sc_scatter(msg2, dst2d, zeros):
    return _sc_kernels()[1](msg2, dst2d, zeros)


def _energy_body(g_ref, ea_ref, et_ref, out_ref):
    i = pl.program_id(0)
    s = (jnp.sum(g_ref[...] * g_ref[...])
         + jnp.sum(ea_ref[...] * ea_ref[...])
         + jnp.sum(et_ref[...] * et_ref[...]))
    tile = jnp.full((8, 128), s, jnp.float32)

    @pl.when(i == 0)
    def _():
        out_ref[...] = jnp.zeros_like(out_ref)

    out_ref[...] += tile


def _energy(gath, ea, et):
    return pl.pallas_call(
        _energy_body,
        grid=(_NEB,),
        in_specs=[
            pl.BlockSpec((_EB, _EMB), lambda i: (i, 0)),
            pl.BlockSpec((_EB, 16), lambda i: (i, 0)),
            pl.BlockSpec((_EB, 16), lambda i: (i, 0)),
        ],
        out_specs=pl.BlockSpec((8, 128), lambda i: (0, 0)),
        out_shape=jax.ShapeDtypeStruct((8, 128), jnp.float32),
        compiler_params=pltpu.CompilerParams(
            dimension_semantics=("arbitrary",)),
    )(gath, ea, et)


def _softshrink(x):
    return jnp.where(x > _LAMBD, x - _LAMBD,
                     jnp.where(x < -_LAMBD, x + _LAMBD, 0.0))


def _main_body(g_ref, ea_ref, et_ref, mr_ref, mi_ref, wr2_ref, wi2_ref,
               b2_ref, rb1_ref, ib1_ref, clo_ref, chi_ref, es_ref, out_ref):
    f32 = jnp.float32
    x = jnp.concatenate([g_ref[...], ea_ref[...], et_ref[...]], axis=1)
    U = jnp.dot(x, mr_ref[...], preferred_element_type=f32)
    V = jnp.dot(x, mi_ref[...], preferred_element_type=f32)
    energy = _IN * jnp.sum(x * x, axis=1, keepdims=True)
    ES = _IN * es_ref[0:1, 0:1]
    yrs, yis = [], []
    for k in range(_NK):
        lo = clo_ref[k:k + 1] * ES
        hi = chi_ref[k:k + 1] * ES
        m = jnp.logical_and(energy >= lo, energy <= hi).astype(f32)
        o_r = m * U[:, k * _EMB:(k + 1) * _EMB] + rb1_ref[k:k + 1, :]
        o_i = m * V[:, k * _EMB:(k + 1) * _EMB] + ib1_ref[k:k + 1, :]
        yrs.append(_softshrink(jnp.maximum(o_r, 0.0)))
        yis.append(_softshrink(jnp.maximum(o_i, 0.0)))
    yr = jnp.concatenate(yrs, axis=1)
    yi = jnp.concatenate(yis, axis=1)
    out = (jnp.dot(yr, wr2_ref[...], preferred_element_type=f32)
           + jnp.dot(yi, wi2_ref[...], preferred_element_type=f32)
           + b2_ref[...])
    row0 = pl.program_id(0) * _EB
    rid = row0 + lax.broadcasted_iota(jnp.int32, (_EB, 1), 0)
    out_ref[...] = jnp.where(rid < _N_EDGES, out, 0.0)


def _main(gath, ea, et, Mr, Mi, Wr2, Wi2, b2, rb1, ib1, clo, chi, esum):
    full = lambda a, b: pl.BlockSpec((a, b), lambda i: (0, 0))
    return pl.pallas_call(
        _main_body,
        grid=(_NEB,),
        in_specs=[
            pl.BlockSpec((_EB, _EMB), lambda i: (i, 0)),
            pl.BlockSpec((_EB, 16), lambda i: (i, 0)),
            pl.BlockSpec((_EB, 16), lambda i: (i, 0)),
            full(_IN, _Y), full(_IN, _Y),
            full(_Y, _EMB), full(_Y, _EMB),
            full(1, _EMB), full(_NK, _EMB), full(_NK, _EMB),
            full(_NK, 1), full(_NK, 1), full(8, 128),
        ],
        out_specs=pl.BlockSpec((_EB, _EMB), lambda i: (i, 0)),
        out_shape=jax.ShapeDtypeStruct((_E_PAD, _EMB), jnp.float32),
        compiler_params=pltpu.CompilerParams(
            dimension_semantics=("arbitrary",)),
    )(gath, ea, et, Mr, Mi, Wr2, Wi2, b2, rb1, ib1, clo, chi, esum)


def _final_body(acc_ref, bc_ref, lw_ref, lb_ref, g_ref, b_ref, out_ref):
    o = acc_ref[0] + acc_ref[1] + bc_ref[...]
    o = jnp.dot(o, lw_ref[...], preferred_element_type=jnp.float32) + lb_ref[...]
    mean = jnp.mean(o, axis=1, keepdims=True)
    d = o - mean
    var = jnp.mean(d * d, axis=1, keepdims=True)
    o = d * lax.rsqrt(var + 1e-5) * g_ref[...] + b_ref[...]
    out_ref[...] = jnp.maximum(o, 0.0)


def _final(acc, bc, lw, lb, g, b):
    full = lambda a, bb: pl.BlockSpec((a, bb), lambda i: (0, 0))
    return pl.pallas_call(
        _final_body,
        grid=(_NNB,),
        in_specs=[
            pl.BlockSpec((2, _NB, _EMB), lambda i: (0, i, 0)),
            pl.BlockSpec((_NB, _EMB), lambda i: (i, 0)),
            full(_EMB, _EMB), full(1, _EMB), full(1, _EMB), full(1, _EMB),
        ],
        out_specs=pl.BlockSpec((_NB, _EMB), lambda i: (i, 0)),
        out_shape=jax.ShapeDtypeStruct((_N_NODES, _EMB), jnp.float32),
        compiler_params=pltpu.CompilerParams(
            dimension_semantics=("arbitrary",)),
    )(acc, bc, lw, lb, g, b)


def kernel(hidden, edge_index, edge_attr, edge_time_emb, boundary_condition,
           alpha, r1, i1, rb1, ib1, fre_W, fre_b, comb_W, comb_b,
           lin_W, lin_b, ln_g, ln_b):
    f32 = jnp.float32
    src = edge_index[0]
    dst = edge_index[1]
    pad = _E_PAD - _N_EDGES
    table = jnp.concatenate([hidden, jnp.zeros((8, _EMB), f32)], axis=0)
    src_p = jnp.concatenate(
        [src, jnp.full((pad,), _N_NODES, src.dtype)]).astype(jnp.int32)
    dst2d = jnp.concatenate(
        [dst, jnp.zeros((pad,), dst.dtype)]).astype(jnp.int32).reshape(-1, _CHUNK)
    ea_p = jnp.concatenate([edge_attr, jnp.zeros((pad, 16), f32)], axis=0)
    et_p = jnp.concatenate([edge_time_emb, jnp.zeros((pad, 16), f32)], axis=0)
    combE = comb_W[0::2]

    Mr, Mi, Wr2, Wi2, b2, clo, chi = _prep(
        r1, i1, fre_W, combE, fre_b.reshape(1, _EMB),
        comb_b.reshape(1, _EMB), alpha)

    gath = _sc_gather(table, src_p)
    esum = _energy(gath, ea_p, et_p)
    msg2 = _main(gath, ea_p, et_p, Mr, Mi, Wr2, Wi2, b2, rb1, ib1,
                 clo, chi, esum)

    zeros = jnp.zeros((_STRIPE, _EMB), f32)
    acc = _sc_scatter(msg2, dst2d, zeros)

    return _final(acc, boundary_condition, lin_W, lin_b.reshape(1, _EMB),
                  ln_g.reshape(1, _EMB), ln_b.reshape(1, _EMB))


# speedup vs baseline: 11.6645x; 1.0850x over previous
import functools

import numpy as np
import jax
import jax.numpy as jnp
from jax import lax
from jax.experimental import pallas as pl
from jax.experimental.pallas import tpu as pltpu
from jax.experimental.pallas import tpu_sc as plsc

_N_NODES = 10000
_N_EDGES = 160000
_EMB = 128
_IN = 160
_NK = 3
_LAMBD = 0.01
_Y = _NK * _EMB

_NW = 32
_CHUNK = 128
_E_PAD = 163840
_CH_PER_W = _E_PAD // (_NW * _CHUNK)
_EDGES_PER_W = _E_PAD // _NW
_STAGE = 8
_STRIPE = 624
_TAIL = _N_NODES - 16 * _STRIPE

_EB = 2048
_NEB = _E_PAD // _EB
_NB = 2000
_NNB = _N_NODES // _NB


def _dft_consts():
    j1 = np.arange(_IN)
    a1 = 2.0 * np.pi * np.outer(j1, j1) / _IN
    C = np.cos(a1).astype(np.float32)
    S = (-np.sin(a1)).astype(np.float32)
    j2 = np.arange(_Y)
    a2 = 2.0 * np.pi * np.outer(j2, j2) / _Y
    A = (np.cos(a2) / _Y).astype(np.float32)
    B = (-np.sin(a2) / _Y).astype(np.float32)
    return C, S, A, B


_C, _S, _A, _B = _dft_consts()
_FAC = np.array([[(2 * k + 1) / (2.0 * _NK)] for k in range(_NK)], np.float32)


def _prep_body(r1, i1, freW, combE, freb, combb, alpha, C, S, A, B,
               mr_o, mi_o, wr2_o, wi2_o, b2_o, clo_o, chi_o):
    hp = lax.Precision.HIGHEST
    R = jnp.concatenate([r1[0], r1[1], r1[2]], axis=1)
    I = jnp.concatenate([i1[0], i1[1], i1[2]], axis=1)
    Cm = C[...]
    Sm = S[...]
    mr_o[...] = (jnp.dot(Cm, R, precision=hp, preferred_element_type=jnp.float32)
                 - jnp.dot(Sm, I, precision=hp, preferred_element_type=jnp.float32))
    mi_o[...] = (jnp.dot(Sm, R, precision=hp, preferred_element_type=jnp.float32)
                 + jnp.dot(Cm, I, precision=hp, preferred_element_type=jnp.float32))
    W2 = jnp.dot(freW[...], combE[...], precision=hp,
                 preferred_element_type=jnp.float32)
    wr2_o[...] = jnp.dot(A[...], W2, precision=hp,
                         preferred_element_type=jnp.float32)
    wi2_o[...] = jnp.dot(B[...], W2, precision=hp,
                         preferred_element_type=jnp.float32)
    b2_o[...] = (jnp.dot(freb[...], combE[...], precision=hp,
                         preferred_element_type=jnp.float32) + combb[...])
    av = alpha[...]
    ik = lax.broadcasted_iota(jnp.int32, (_NK, 1), 0).astype(jnp.float32)
    fac = (2.0 * ik + 1.0) / (2.0 * _NK)
    half = 1.0 / (2.0 * _NK)
    clo_o[...] = av * fac - half / av
    chi_o[...] = av * fac + half / av


def _prep(r1, i1, freW, combE, freb, combb, alpha):
    f32 = jnp.float32
    outs = (
        jax.ShapeDtypeStruct((_IN, _Y), f32),
        jax.ShapeDtypeStruct((_IN, _Y), f32),
        jax.ShapeDtypeStruct((_Y, _EMB), f32),
        jax.ShapeDtypeStruct((_Y, _EMB), f32),
        jax.ShapeDtypeStruct((1, _EMB), f32),
        jax.ShapeDtypeStruct((_NK, 1), f32),
        jax.ShapeDtypeStruct((_NK, 1), f32),
    )
    return pl.pallas_call(_prep_body, out_shape=outs)(
        r1, i1, freW, combE, freb, combb, alpha,
        jnp.asarray(_C), jnp.asarray(_S), jnp.asarray(_A), jnp.asarray(_B))


@functools.cache
def _sc_kernels():
    mesh = plsc.VectorSubcoreMesh(core_axis_name="c", subcore_axis_name="s")

    @functools.partial(
        pl.kernel,
        out_type=jax.ShapeDtypeStruct((_E_PAD, _EMB), jnp.float32),
        mesh=mesh,
        scratch_types=[
            pltpu.VMEM((_CH_PER_W, _CHUNK), jnp.int32),
            pltpu.VMEM((2, 2 * _CHUNK, _EMB), jnp.float32),
            pltpu.SemaphoreType.DMA,
            pltpu.SemaphoreType.DMA,
            pltpu.SemaphoreType.DMA,
            pltpu.SemaphoreType.DMA,
        ],
    )
    def sc_gather(table_hbm, src2d_hbm, out_hbm, idx_all, rows_v,
                  sg0, sg1, so0, so1):
        c = lax.axis_index("c")
        s = lax.axis_index("s")
        wid = s * 2 + c
        semg = (sg0, sg1)
        semo = (so0, so1)
        irow = pl.multiple_of(wid * _CH_PER_W, 8)
        pltpu.sync_copy(src2d_hbm.at[pl.ds(irow, _CH_PER_W)], idx_all)
        nround = _CH_PER_W // 2

        @pl.loop(0, nround, step=2)
        def _(r):
            for db in range(2):
                rr = r + db
                buf = rows_v.at[db]

                @pl.when(rr >= 2)
                def _():
                    pltpu.make_async_copy(
                        buf, out_hbm.at[pl.ds(0, 2 * _CHUNK)], semo[db]
                    ).wait()

                d0 = pltpu.async_copy(
                    table_hbm.at[idx_all.at[2 * rr]],
                    buf.at[pl.ds(0, _CHUNK)], semg[db])
                d1 = pltpu.async_copy(
                    table_hbm.at[idx_all.at[2 * rr + 1]],
                    buf.at[pl.ds(_CHUNK, _CHUNK)], semg[db])
                d0.wait()
                d1.wait()
                base = pl.multiple_of(wid * _EDGES_PER_W, _CHUNK) + rr * (2 * _CHUNK)
                pltpu.async_copy(buf, out_hbm.at[pl.ds(base, 2 * _CHUNK)],
                                 semo[db])

        for db in range(2):
            pltpu.make_async_copy(
                rows_v.at[db], out_hbm.at[pl.ds(0, 2 * _CHUNK)], semo[db]
            ).wait()

    @functools.partial(
        pl.kernel,
        out_type=jax.ShapeDtypeStruct((2, _N_NODES, _EMB), jnp.float32),
        mesh=mesh,
        scratch_types=[
            pltpu.VMEM((_CH_PER_W, _CHUNK), jnp.int32),
            pltpu.VMEM((2, _CHUNK, _EMB), jnp.float32),
            pltpu.VMEM_SHARED((_N_NODES, _EMB), jnp.float32),
            pltpu.SemaphoreType.DMA,
            pltpu.SemaphoreType.DMA,
            pltpu.SemaphoreType.DMA,
            pltpu.SemaphoreType.DMA,
        ],
    )
    def sc_scatter(msg_hbm, dst2d_hbm, zeros_hbm, out_hbm, idx_all, vals_v,
                   acc_sh, sv0, sv1, ss0, ss1):
        c = lax.axis_index("c")
        s = lax.axis_index("s")
        wid = s * 2 + c
        semv = (sv0, sv1)
        sems = (ss0, ss1)
        srow = pl.multiple_of(s * _STRIPE, 8)
        pltpu.sync_copy(zeros_hbm.at[pl.ds(0, _STRIPE)],
                        acc_sh.at[pl.ds(srow, _STRIPE)])

        @pl.when(s == 15)
        def _():
            pltpu.sync_copy(zeros_hbm.at[pl.ds(0, _TAIL)],
                            acc_sh.at[pl.ds(16 * _STRIPE, _TAIL)])

        irow = pl.multiple_of(wid * _CH_PER_W, 8)
        pltpu.sync_copy(dst2d_hbm.at[pl.ds(irow, _CH_PER_W)], idx_all)
        plsc.subcore_barrier()

        @pl.loop(0, _CH_PER_W, step=2)
        def _(r):
            for db in range(2):
                rr = r + db
                buf = vals_v.at[db]

                @pl.when(rr >= 2)
                def _():
                    pltpu.make_async_copy(
                        buf, acc_sh.at[idx_all.at[0]], sems[db]).wait()

                base = pl.multiple_of(wid * _EDGES_PER_W, _CHUNK) + rr * _CHUNK
                pltpu.async_copy(msg_hbm.at[pl.ds(base, _CHUNK)], buf,
                                 semv[db]).wait()
                pltpu.async_copy(buf, acc_sh.at[idx_all.at[rr]],
                                 sems[db], add=True)

        for db in range(2):
            pltpu.make_async_copy(
                vals_v.at[db], acc_sh.at[idx_all.at[0]], sems[db]).wait()
        plsc.subcore_barrier()
        pltpu.sync_copy(acc_sh.at[pl.ds(srow, _STRIPE)],
                        out_hbm.at[c, pl.ds(srow, _STRIPE)])

        @pl.when(s == 15)
        def _():
            pltpu.sync_copy(acc_sh.at[pl.ds(16 * _STRIPE, _TAIL)],
                            out_hbm.at[c, pl.ds(16 * _STRIPE, _TAIL)])

    return sc_gather, sc_scatter


def _sc_gather(table, src_p):
    return _sc_kernels()[0](table, src_p)


def _sc_scatter(msg2, dst2d, zeros):
    return _sc_kernels()[1](msg2, dst2d, zeros)


def _energy_body(g_ref, ea_ref, et_ref, out_ref):
    i = pl.program_id(0)
    s = (jnp.sum(g_ref[...] * g_ref[...])
         + jnp.sum(ea_ref[...] * ea_ref[...])
         + jnp.sum(et_ref[...] * et_ref[...]))
    tile = jnp.full((8, 128), s, jnp.float32)

    @pl.when(i == 0)
    def _():
        out_ref[...] = jnp.zeros_like(out_ref)

    out_ref[...] += tile


def _energy(gath, ea, et):
    return pl.pallas_call(
        _energy_body,
        grid=(_NEB,),
        in_specs=[
            pl.BlockSpec((_EB, _EMB), lambda i: (i, 0)),
            pl.BlockSpec((_EB, 16), lambda i: (i, 0)),
            pl.BlockSpec((_EB, 16), lambda i: (i, 0)),
        ],
        out_specs=pl.BlockSpec((8, 128), lambda i: (0, 0)),
        out_shape=jax.ShapeDtypeStruct((8, 128), jnp.float32),
        compiler_params=pltpu.CompilerParams(
            dimension_semantics=("arbitrary",)),
    )(gath, ea, et)


def _softshrink(x):
    return jnp.where(x > _LAMBD, x - _LAMBD,
                     jnp.where(x < -_LAMBD, x + _LAMBD, 0.0))


def _main_body(g_ref, ea_ref, et_ref, mr_ref, mi_ref, wr2_ref, wi2_ref,
               b2_ref, rb1_ref, ib1_ref, clo_ref, chi_ref, es_ref, out_ref):
    f32 = jnp.float32
    x = jnp.concatenate([g_ref[...], ea_ref[...], et_ref[...]], axis=1)
    U = jnp.dot(x, mr_ref[...], preferred_element_type=f32)
    V = jnp.dot(x, mi_ref[...], preferred_element_type=f32)
    energy = _IN * jnp.sum(x * x, axis=1, keepdims=True)
    ES = _IN * es_ref[0:1, 0:1]
    yrs, yis = [], []
    for k in range(_NK):
        lo = clo_ref[k:k + 1] * ES
        hi = chi_ref[k:k + 1] * ES
        m = jnp.logical_and(energy >= lo, energy <= hi).astype(f32)
        o_r = m * U[:, k * _EMB:(k + 1) * _EMB] + rb1_ref[k:k + 1, :]
        o_i = m * V[:, k * _EMB:(k + 1) * _EMB] + ib1_ref[k:k + 1, :]
        yrs.append(_softshrink(jnp.maximum(o_r, 0.0)))
        yis.append(_softshrink(jnp.maximum(o_i, 0.0)))
    yr = jnp.concatenate(yrs, axis=1)
    yi = jnp.concatenate(yis, axis=1)
    out = (jnp.dot(yr, wr2_ref[...], preferred_element_type=f32)
           + jnp.dot(yi, wi2_ref[...], preferred_element_type=f32)
           + b2_ref[...])
    row0 = pl.program_id(0) * _EB
    rid = row0 + lax.broadcasted_iota(jnp.int32, (_EB, 1), 0)
    out_ref[...] = jnp.where(rid < _N_EDGES, out, 0.0)


def _main(gath, ea, et, Mr, Mi, Wr2, Wi2, b2, rb1, ib1, clo, chi, esum):
    full = lambda a, b: pl.BlockSpec((a, b), lambda i: (0, 0))
    return pl.pallas_call(
        _main_body,
        grid=(_NEB,),
        in_specs=[
            pl.BlockSpec((_EB, _EMB), lambda i: (i, 0)),
            pl.BlockSpec((_EB, 16), lambda i: (i, 0)),
            pl.BlockSpec((_EB, 16), lambda i: (i, 0)),
            full(_IN, _Y), full(_IN, _Y),
            full(_Y, _EMB), full(_Y, _EMB),
            full(1, _EMB), full(_NK, _EMB), full(_NK, _EMB),
            full(_NK, 1), full(_NK, 1), full(8, 128),
        ],
        out_specs=pl.BlockSpec((_EB, _EMB), lambda i: (i, 0)),
        out_shape=jax.ShapeDtypeStruct((_E_PAD, _EMB), jnp.float32),
        compiler_params=pltpu.CompilerParams(
            dimension_semantics=("arbitrary",)),
    )(gath, ea, et, Mr, Mi, Wr2, Wi2, b2, rb1, ib1, clo, chi, esum)


def _final_body(acc_ref, bc_ref, lw_ref, lb_ref, g_ref, b_ref, out_ref):
    o = acc_ref[0] + acc_ref[1] + bc_ref[...]
    o = jnp.dot(o, lw_ref[...], preferred_element_type=jnp.float32) + lb_ref[...]
    mean = jnp.mean(o, axis=1, keepdims=True)
    d = o - mean
    var = jnp.mean(d * d, axis=1, keepdims=True)
    o = d * lax.rsqrt(var + 1e-5) * g_ref[...] + b_ref[...]
    out_ref[...] = jnp.maximum(o, 0.0)


def _final(acc, bc, lw, lb, g, b):
    full = lambda a, bb: pl.BlockSpec((a, bb), lambda i: (0, 0))
    return pl.pallas_call(
        _final_body,
        grid=(_NNB,),
        in_specs=[
            pl.BlockSpec((2, _NB, _EMB), lambda i: (0, i, 0)),
            pl.BlockSpec((_NB, _EMB), lambda i: (i, 0)),
            full(_EMB, _EMB), full(1, _EMB), full(1, _EMB), full(1, _EMB),
        ],
        out_specs=pl.BlockSpec((_NB, _EMB), lambda i: (i, 0)),
        out_shape=jax.ShapeDtypeStruct((_N_NODES, _EMB), jnp.float32),
        compiler_params=pltpu.CompilerParams(
            dimension_semantics=("arbitrary",)),
    )(acc, bc, lw, lb, g, b)


def kernel(hidden, edge_index, edge_attr, edge_time_emb, boundary_condition,
           alpha, r1, i1, rb1, ib1, fre_W, fre_b, comb_W, comb_b,
           lin_W, lin_b, ln_g, ln_b):
    f32 = jnp.float32
    src = edge_index[0]
    dst = edge_index[1]
    pad = _E_PAD - _N_EDGES
    table = jnp.concatenate([hidden, jnp.zeros((8, _EMB), f32)], axis=0)
    src2d = jnp.concatenate(
        [src, jnp.full((pad,), _N_NODES, src.dtype)]
    ).astype(jnp.int32).reshape(-1, _CHUNK)
    dst2d = jnp.concatenate(
        [dst, jnp.zeros((pad,), dst.dtype)]).astype(jnp.int32).reshape(-1, _CHUNK)
    ea_p = jnp.concatenate([edge_attr, jnp.zeros((pad, 16), f32)], axis=0)
    et_p = jnp.concatenate([edge_time_emb, jnp.zeros((pad, 16), f32)], axis=0)
    combE = comb_W[0::2]

    Mr, Mi, Wr2, Wi2, b2, clo, chi = _prep(
        r1, i1, fre_W, combE, fre_b.reshape(1, _EMB),
        comb_b.reshape(1, _EMB), alpha)

    gath = _sc_gather(table, src2d)
    esum = _energy(gath, ea_p, et_p)
    msg2 = _main(gath, ea_p, et_p, Mr, Mi, Wr2, Wi2, b2, rb1, ib1,
                 clo, chi, esum)

    zeros = jnp.zeros((_STRIPE, _EMB), f32)
    acc = _sc_scatter(msg2, dst2d, zeros)

    return _final(acc, boundary_condition, lin_W, lin_b.reshape(1, _EMB),
                  ln_g.reshape(1, _EMB), ln_b.reshape(1, _EMB))
